# baseline (speedup 1.0000x reference)
import jax
import jax.numpy as jnp
from jax.experimental import pallas as pl


def _mm_small_kernel(x_ref, w_ref, o_ref):
    o_ref[...] = jnp.dot(x_ref[...], w_ref[...],
                         preferred_element_type=jnp.float32)


def _small_matmul(x, w):
    return pl.pallas_call(
        _mm_small_kernel,
        out_shape=jax.ShapeDtypeStruct((x.shape[0], w.shape[1]), jnp.float32),
    )(x, w)


def _spmm_row_kernel(a_ref, z_ref, o_ref):
    o_ref[...] = jnp.dot(a_ref[...], z_ref[...],
                         preferred_element_type=jnp.float32)


def _big_matmul(a, z, bm):
    n = a.shape[0]
    d = z.shape[1]
    return pl.pallas_call(
        _spmm_row_kernel,
        grid=(n // bm,),
        in_specs=[
            pl.BlockSpec((bm, n), lambda i: (i, 0)),
            pl.BlockSpec((n, d), lambda i: (0, 0)),
        ],
        out_specs=pl.BlockSpec((bm, d), lambda i: (i, 0)),
        out_shape=jax.ShapeDtypeStruct((n, d), jnp.float32),
    )(a, z)


def kernel(X, A_hat, W1, W2):
    n = A_hat.shape[0]
    bm = 400 if n % 400 == 0 else n
    z1 = _small_matmul(X, W1)
    h = _big_matmul(A_hat, z1, bm)
    t = _small_matmul(h, W2)
    y = _big_matmul(A_hat, t, bm)
    return y


# speedup vs baseline: 1.0101x; 1.0101x over previous
import jax
import jax.numpy as jnp
from jax.experimental import pallas as pl


def _mm_small_kernel(x_ref, w_ref, o_ref):
    o_ref[...] = jnp.dot(x_ref[...], w_ref[...],
                         preferred_element_type=jnp.float32
                         ).astype(jnp.bfloat16)


def _small_matmul(x, w):
    return pl.pallas_call(
        _mm_small_kernel,
        out_shape=jax.ShapeDtypeStruct((x.shape[0], w.shape[1]),
                                       jnp.bfloat16),
    )(x, w)


def _spmm_row_kernel(a_ref, z_ref, o_ref):
    a16 = a_ref[...].astype(jnp.bfloat16)
    o_ref[...] = jnp.dot(a16, z_ref[...],
                         preferred_element_type=jnp.float32)


def _big_matmul(a, z, bm):
    n = a.shape[0]
    d = z.shape[1]
    return pl.pallas_call(
        _spmm_row_kernel,
        grid=(n // bm,),
        in_specs=[
            pl.BlockSpec((bm, n), lambda i: (i, 0)),
            pl.BlockSpec((n, d), lambda i: (0, 0)),
        ],
        out_specs=pl.BlockSpec((bm, d), lambda i: (i, 0)),
        out_shape=jax.ShapeDtypeStruct((n, d), jnp.float32),
    )(a, z)


def kernel(X, A_hat, W1, W2):
    n = A_hat.shape[0]
    bm = 400 if n % 400 == 0 else n
    z1 = _small_matmul(X, W1)
    h = _big_matmul(A_hat, z1, bm)
    t = _small_matmul(h, W2)
    y = _big_matmul(A_hat, t, bm)
    return y


# speedup vs baseline: 1.0475x; 1.0370x over previous
import functools

import jax
import jax.numpy as jnp
from jax.experimental import pallas as pl
from jax.experimental.pallas import tpu as pltpu


def _fused_pass_kernel(x_ref, w_ref, a_ref, o_ref, z_ref, *, out_bf16):
    @pl.when(pl.program_id(0) == 0)
    def _():
        z_ref[...] = jnp.dot(
            x_ref[...].astype(jnp.bfloat16),
            w_ref[...].astype(jnp.bfloat16),
            preferred_element_type=jnp.float32,
        ).astype(jnp.bfloat16)

    acc = jnp.dot(a_ref[...].astype(jnp.bfloat16), z_ref[...],
                  preferred_element_type=jnp.float32)
    o_ref[...] = acc.astype(jnp.bfloat16) if out_bf16 else acc


def _fused_pass(a, x, w, bm, out_bf16):
    n = a.shape[0]
    d = w.shape[1]
    out_dtype = jnp.bfloat16 if out_bf16 else jnp.float32
    return pl.pallas_call(
        functools.partial(_fused_pass_kernel, out_bf16=out_bf16),
        grid=(n // bm,),
        in_specs=[
            pl.BlockSpec((x.shape[0], d), lambda i: (0, 0)),
            pl.BlockSpec((d, d), lambda i: (0, 0)),
            pl.BlockSpec((bm, n), lambda i: (i, 0)),
        ],
        out_specs=pl.BlockSpec((bm, d), lambda i: (i, 0)),
        out_shape=jax.ShapeDtypeStruct((n, d), out_dtype),
        scratch_shapes=[pltpu.VMEM((n, d), jnp.bfloat16)],
    )(x, w, a)


def kernel(X, A_hat, W1, W2):
    n = A_hat.shape[0]
    bm = 400 if n % 400 == 0 else n
    h = _fused_pass(A_hat, X, W1, bm, out_bf16=True)
    y = _fused_pass(A_hat, h, W2, bm, out_bf16=False)
    return y
